# baseline (speedup 1.0000x reference)
import jax
import jax.numpy as jnp
from jax import lax
from jax.experimental import pallas as pl
from jax.experimental.pallas import tpu as pltpu

_HALF_ACC = 256
_HALF_IN = 41920
_BK = 2096
_K_TILES = _HALF_IN // _BK

_DNT = (((0,), (0,)), ((), ()))
_DNS = (((1,), (0,)), ((), ()))


def _nn_body(wf_ref, bf_ref, stm_ref, Ww_ref, Wb_ref,
             bw_ref, bb_ref, W1_ref, b1_ref, W2_ref, b2_ref, Wo_ref, bo_ref,
             out_ref, accw_ref, accb_ref):
    k = pl.program_id(0)

    pw = lax.dot_general(Ww_ref[...], wf_ref[...], _DNT,
                         preferred_element_type=jnp.float32)
    pb = lax.dot_general(Wb_ref[...], bf_ref[...], _DNT,
                         preferred_element_type=jnp.float32)

    @pl.when(k == 0)
    def _init():
        accw_ref[...] = pw
        accb_ref[...] = pb

    @pl.when(k > 0)
    def _accum():
        accw_ref[...] += pw
        accb_ref[...] += pb

    @pl.when(k == _K_TILES - 1)
    def _tail():
        accw = accw_ref[...] + bw_ref[...]
        accb = accb_ref[...] + bb_ref[...]
        stm = stm_ref[...]
        h1 = jnp.clip((1.0 - stm) * accw + stm * accb, 0.0, 1.0)
        h2 = jnp.clip(stm * accw + (1.0 - stm) * accb, 0.0, 1.0)
        W1 = W1_ref[...]
        o1 = (lax.dot_general(W1[:, :_HALF_ACC], h1, _DNS,
                              preferred_element_type=jnp.float32)
              + lax.dot_general(W1[:, _HALF_ACC:], h2, _DNS,
                                preferred_element_type=jnp.float32)
              + b1_ref[...])
        i2 = jnp.clip(o1, 0.0, 1.0)
        o2 = lax.dot_general(W2_ref[...], i2, _DNS,
                             preferred_element_type=jnp.float32) + b2_ref[...]
        io = jnp.clip(o2, 0.0, 1.0)
        out_ref[...] = lax.dot_general(Wo_ref[...], io, _DNS,
                                       preferred_element_type=jnp.float32) + bo_ref[0]


def kernel(white_features, black_features, stm, Ww, bw, Wb, bb,
           W1, b1, W2, b2, Wo, bo):
    batch = white_features.shape[0]
    out = pl.pallas_call(
        _nn_body,
        grid=(_K_TILES,),
        in_specs=[
            pl.BlockSpec((_BK, batch), lambda k: (k, 0)),
            pl.BlockSpec((_BK, batch), lambda k: (k, 0)),
            pl.BlockSpec((1, batch), lambda k: (0, 0)),
            pl.BlockSpec((_BK, _HALF_ACC), lambda k: (k, 0)),
            pl.BlockSpec((_BK, _HALF_ACC), lambda k: (k, 0)),
            pl.BlockSpec((_HALF_ACC, 1), lambda k: (0, 0)),
            pl.BlockSpec((_HALF_ACC, 1), lambda k: (0, 0)),
            pl.BlockSpec((32, 2 * _HALF_ACC), lambda k: (0, 0)),
            pl.BlockSpec((32, 1), lambda k: (0, 0)),
            pl.BlockSpec((32, 32), lambda k: (0, 0)),
            pl.BlockSpec((32, 1), lambda k: (0, 0)),
            pl.BlockSpec((128, 32), lambda k: (0, 0)),
            pl.BlockSpec(memory_space=pltpu.SMEM),
        ],
        out_specs=pl.BlockSpec((128, batch), lambda k: (0, 0)),
        out_shape=jax.ShapeDtypeStruct((128, batch), jnp.float32),
        scratch_shapes=[
            pltpu.VMEM((_HALF_ACC, batch), jnp.float32),
            pltpu.VMEM((_HALF_ACC, batch), jnp.float32),
        ],
        compiler_params=pltpu.CompilerParams(
            dimension_semantics=("arbitrary",),
            vmem_limit_bytes=60 * 1024 * 1024,
        ),
    )(white_features.T, black_features.T, stm.T, Ww.T, Wb.T,
      bw.reshape(-1, 1), bb.reshape(-1, 1),
      W1, b1.reshape(-1, 1), W2, b2.reshape(-1, 1),
      jnp.pad(Wo, ((0, 128 - Wo.shape[0]), (0, 0))), bo)
    return out[:1, :].T


# speedup vs baseline: 1.0376x; 1.0376x over previous
import jax
import jax.numpy as jnp
from jax import lax
from jax.experimental import pallas as pl
from jax.experimental.pallas import tpu as pltpu

_HALF_ACC = 256
_HALF_IN = 41920
_BK = 2096
_K_TILES = _HALF_IN // _BK

_DNT = (((0,), (0,)), ((), ()))
_DN = (((1,), (1,)), ((), ()))


def _nn_body(wf_ref, bf_ref, stm_ref, Ww_ref, Wb_ref,
             bw_ref, bb_ref, W1_ref, b1_ref, W2_ref, b2_ref, Wo_ref, bo_ref,
             out_ref, accw_ref, accb_ref):
    k = pl.program_id(0)

    pw = lax.dot_general(wf_ref[...], Ww_ref[...], _DNT,
                         preferred_element_type=jnp.float32)
    pb = lax.dot_general(bf_ref[...], Wb_ref[...], _DNT,
                         preferred_element_type=jnp.float32)

    @pl.when(k == 0)
    def _init():
        accw_ref[...] = pw
        accb_ref[...] = pb

    @pl.when(k > 0)
    def _accum():
        accw_ref[...] += pw
        accb_ref[...] += pb

    @pl.when(k == _K_TILES - 1)
    def _tail():
        accw = accw_ref[...] + bw_ref[...]
        accb = accb_ref[...] + bb_ref[...]
        stm = stm_ref[...]
        h1 = jnp.clip((1.0 - stm) * accw + stm * accb, 0.0, 1.0)
        h2 = jnp.clip(stm * accw + (1.0 - stm) * accb, 0.0, 1.0)
        W1 = W1_ref[...]
        o1 = (lax.dot_general(h1, W1[:, :_HALF_ACC], _DN,
                              preferred_element_type=jnp.float32)
              + lax.dot_general(h2, W1[:, _HALF_ACC:], _DN,
                                preferred_element_type=jnp.float32)
              + b1_ref[...])
        i2 = jnp.clip(o1, 0.0, 1.0)
        o2 = lax.dot_general(i2, W2_ref[...], _DN,
                             preferred_element_type=jnp.float32) + b2_ref[...]
        io = jnp.clip(o2, 0.0, 1.0)
        out_ref[...] = lax.dot_general(Wo_ref[...], io, _DN,
                                       preferred_element_type=jnp.float32) + bo_ref[0]


def kernel(white_features, black_features, stm, Ww, bw, Wb, bb,
           W1, b1, W2, b2, Wo, bo):
    batch = white_features.shape[0]
    out = pl.pallas_call(
        _nn_body,
        grid=(_K_TILES,),
        in_specs=[
            pl.BlockSpec((_BK, batch), lambda k: (k, 0)),
            pl.BlockSpec((_BK, batch), lambda k: (k, 0)),
            pl.BlockSpec((batch, 1), lambda k: (0, 0)),
            pl.BlockSpec((_BK, _HALF_ACC), lambda k: (k, 0)),
            pl.BlockSpec((_BK, _HALF_ACC), lambda k: (k, 0)),
            pl.BlockSpec((1, _HALF_ACC), lambda k: (0, 0)),
            pl.BlockSpec((1, _HALF_ACC), lambda k: (0, 0)),
            pl.BlockSpec((32, 2 * _HALF_ACC), lambda k: (0, 0)),
            pl.BlockSpec((1, 32), lambda k: (0, 0)),
            pl.BlockSpec((32, 32), lambda k: (0, 0)),
            pl.BlockSpec((1, 32), lambda k: (0, 0)),
            pl.BlockSpec((8, 32), lambda k: (0, 0)),
            pl.BlockSpec(memory_space=pltpu.SMEM),
        ],
        out_specs=pl.BlockSpec((8, batch), lambda k: (0, 0)),
        out_shape=jax.ShapeDtypeStruct((8, batch), jnp.float32),
        scratch_shapes=[
            pltpu.VMEM((batch, _HALF_ACC), jnp.float32),
            pltpu.VMEM((batch, _HALF_ACC), jnp.float32),
        ],
        compiler_params=pltpu.CompilerParams(
            dimension_semantics=("arbitrary",),
            vmem_limit_bytes=60 * 1024 * 1024,
        ),
    )(white_features.T, black_features.T, stm, Ww.T, Wb.T,
      bw.reshape(1, -1), bb.reshape(1, -1),
      W1, b1.reshape(1, -1), W2, b2.reshape(1, -1),
      jnp.pad(Wo, ((0, 8 - Wo.shape[0]), (0, 0))), bo)
    return out[:1, :].T


# speedup vs baseline: 1.0584x; 1.0201x over previous
import jax
import jax.numpy as jnp
from jax import lax
from jax.experimental import pallas as pl
from jax.experimental.pallas import tpu as pltpu

_HALF_ACC = 256
_HALF_IN = 41920
_BK = 2096
_K_TILES = _HALF_IN // _BK

_DNT = (((0,), (0,)), ((), ()))
_DN = (((1,), (1,)), ((), ()))


def _nn_body(wf_ref, bf_ref, stm_ref, Ww_ref, Wb_ref,
             bw_ref, bb_ref, W1_ref, b1_ref, W2_ref, b2_ref, Wo_ref, bo_ref,
             out_ref, accw_ref, accb_ref):
    k = pl.program_id(0)

    pw = lax.dot_general(wf_ref[...], Ww_ref[...], _DNT,
                         preferred_element_type=jnp.float32)
    pb = lax.dot_general(bf_ref[...], Wb_ref[...], _DNT,
                         preferred_element_type=jnp.float32)

    @pl.when(k == 0)
    def _init():
        accw_ref[...] = pw
        accb_ref[...] = pb

    @pl.when(k > 0)
    def _accum():
        accw_ref[...] += pw
        accb_ref[...] += pb

    @pl.when(k == _K_TILES - 1)
    def _tail():
        accw = accw_ref[...] + bw_ref[...]
        accb = accb_ref[...] + bb_ref[...]
        stm = stm_ref[...].T
        h1 = jnp.clip((1.0 - stm) * accw + stm * accb, 0.0, 1.0)
        h2 = jnp.clip(stm * accw + (1.0 - stm) * accb, 0.0, 1.0)
        W1 = W1_ref[...]
        o1 = (lax.dot_general(h1, W1[:, :_HALF_ACC], _DN,
                              preferred_element_type=jnp.float32)
              + lax.dot_general(h2, W1[:, _HALF_ACC:], _DN,
                                preferred_element_type=jnp.float32)
              + b1_ref[...])
        i2 = jnp.clip(o1, 0.0, 1.0)
        o2 = lax.dot_general(i2, W2_ref[...], _DN,
                             preferred_element_type=jnp.float32) + b2_ref[...]
        io = jnp.clip(o2, 0.0, 1.0)
        wo8 = jnp.broadcast_to(Wo_ref[...], (8, 32))
        out_ref[...] = lax.dot_general(wo8, io, _DN,
                                       preferred_element_type=jnp.float32) + bo_ref[0]


def kernel(white_features, black_features, stm, Ww, bw, Wb, bb,
           W1, b1, W2, b2, Wo, bo):
    batch = white_features.shape[0]
    out = pl.pallas_call(
        _nn_body,
        grid=(_K_TILES,),
        in_specs=[
            pl.BlockSpec((_BK, batch), lambda k: (k, 0)),
            pl.BlockSpec((_BK, batch), lambda k: (k, 0)),
            pl.BlockSpec((1, batch), lambda k: (0, 0)),
            pl.BlockSpec((_BK, _HALF_ACC), lambda k: (k, 0)),
            pl.BlockSpec((_BK, _HALF_ACC), lambda k: (k, 0)),
            pl.BlockSpec((1, _HALF_ACC), lambda k: (0, 0)),
            pl.BlockSpec((1, _HALF_ACC), lambda k: (0, 0)),
            pl.BlockSpec((32, 2 * _HALF_ACC), lambda k: (0, 0)),
            pl.BlockSpec((1, 32), lambda k: (0, 0)),
            pl.BlockSpec((32, 32), lambda k: (0, 0)),
            pl.BlockSpec((1, 32), lambda k: (0, 0)),
            pl.BlockSpec((1, 32), lambda k: (0, 0)),
            pl.BlockSpec(memory_space=pltpu.SMEM),
        ],
        out_specs=pl.BlockSpec((8, batch), lambda k: (0, 0)),
        out_shape=jax.ShapeDtypeStruct((8, batch), jnp.float32),
        scratch_shapes=[
            pltpu.VMEM((batch, _HALF_ACC), jnp.float32),
            pltpu.VMEM((batch, _HALF_ACC), jnp.float32),
        ],
        compiler_params=pltpu.CompilerParams(
            dimension_semantics=("arbitrary",),
            vmem_limit_bytes=60 * 1024 * 1024,
        ),
    )(white_features.T, black_features.T, stm.T, Ww.T, Wb.T,
      bw.reshape(1, -1), bb.reshape(1, -1),
      W1, b1.reshape(1, -1), W2, b2.reshape(1, -1), Wo, bo)
    return out[:1, :].T
